# Initial kernel scaffold; baseline (speedup 1.0000x reference)
import jax
import jax.numpy as jnp
from jax.experimental import pallas as pl


def kernel(input, edge_index, W0, b0, Wc1, bc1, Wc2, bc2):
    raise NotImplementedError("write your pallas kernel here")


# speedup vs baseline: 15.0455x; 15.0455x over previous
import functools

import jax
import jax.numpy as jnp
from jax import lax
from jax.experimental import pallas as pl
from jax.experimental.pallas import tpu as pltpu
from jax.experimental.pallas import tpu_sc as plsc

N = 10000
D = 128
NR = 10240
NC = 2
NS = 16
NW = NC * NS
WIN = 128
RPT = NR // NS
BM = 1024
GRID = NR // BM


def _fill_2d(ref, rows, value):
    v = jnp.full((16,), value, jnp.float32)
    cols = ref.shape[1]

    def body(r, _):
        for j in range(cols // 16):
            ref[r, pl.ds(j * 16, 16)] = v
        return 0

    lax.fori_loop(0, rows, body, 0)


def _sc_deg(dst_pad, epw, nwin):
    @functools.partial(
        pl.kernel,
        mesh=plsc.VectorSubcoreMesh(core_axis_name="c", subcore_axis_name="s"),
        out_type=jax.ShapeDtypeStruct((NC * NR, D), jnp.float32),
        scratch_types=[
            pltpu.VMEM((WIN,), jnp.int32),
            pltpu.VMEM((WIN, D), jnp.float32),
            pltpu.VMEM_SHARED((NR, D), jnp.float32),
        ],
    )
    def k(dst_hbm, out_hbm, didx, ones_v, acc):
        c = lax.axis_index("c")
        s = lax.axis_index("s")
        wid = c * NS + s
        _fill_2d(ones_v, WIN, 0.0)
        for j in range(RPT // WIN):
            pltpu.sync_copy(ones_v, acc.at[pl.ds(s * RPT + j * WIN, WIN)])
        _fill_2d(ones_v, WIN, 1.0)
        plsc.subcore_barrier()
        ebase = wid * epw

        def win(w, _):
            pltpu.sync_copy(dst_hbm.at[pl.ds(ebase + w * WIN, WIN)], didx)
            pltpu.sync_copy(ones_v, acc.at[didx], add=True)
            return 0

        lax.fori_loop(0, nwin, win, 0)
        plsc.subcore_barrier()
        pltpu.sync_copy(acc.at[pl.ds(s * RPT, RPT)],
                        out_hbm.at[pl.ds(c * NR + s * RPT, RPT)])

    return k(dst_pad)


def _sc_scatter(h_pad, src_pad, dst_pad, epw, nwin):
    @functools.partial(
        pl.kernel,
        mesh=plsc.VectorSubcoreMesh(core_axis_name="c", subcore_axis_name="s"),
        out_type=jax.ShapeDtypeStruct((NC * NR, D), jnp.float32),
        scratch_types=[
            pltpu.VMEM((WIN,), jnp.int32),
            pltpu.VMEM((WIN,), jnp.int32),
            pltpu.VMEM((WIN, D), jnp.float32),
            pltpu.VMEM_SHARED((NR, D), jnp.float32),
            pltpu.SemaphoreType.DMA,
        ],
    )
    def k(h_hbm, src_hbm, dst_hbm, out_hbm, sidx, didx, rows, acc, sem):
        c = lax.axis_index("c")
        s = lax.axis_index("s")
        wid = c * NS + s
        _fill_2d(rows, WIN, 0.0)
        for j in range(RPT // WIN):
            pltpu.sync_copy(rows, acc.at[pl.ds(s * RPT + j * WIN, WIN)])
        plsc.subcore_barrier()
        ebase = wid * epw

        def win(w, _):
            b = ebase + w * WIN
            pltpu.sync_copy(src_hbm.at[pl.ds(b, WIN)], sidx)
            pltpu.sync_copy(dst_hbm.at[pl.ds(b, WIN)], didx)
            pltpu.async_copy(h_hbm.at[sidx], rows, sem).wait()
            pltpu.sync_copy(rows, acc.at[didx], add=True)
            return 0

        lax.fori_loop(0, nwin, win, 0)
        plsc.subcore_barrier()
        pltpu.sync_copy(acc.at[pl.ds(s * RPT, RPT)],
                        out_hbm.at[pl.ds(c * NR + s * RPT, RPT)])

    return k(h_pad, src_pad, dst_pad)


def _tc_dis(deg_parts):
    def kfn(d0_ref, d1_ref, oref):
        deg = d0_ref[:, 0:1] + d1_ref[:, 0:1] + 1.0
        oref[...] = jnp.broadcast_to(lax.rsqrt(deg), (BM, D))

    return pl.pallas_call(
        kfn,
        grid=(GRID,),
        in_specs=[
            pl.BlockSpec((BM, D), lambda i: (i, 0)),
            pl.BlockSpec((BM, D), lambda i: (i + GRID, 0)),
        ],
        out_specs=pl.BlockSpec((BM, D), lambda i: (i, 0)),
        out_shape=jax.ShapeDtypeStruct((NR, D), jnp.float32),
    )(deg_parts, deg_parts)


def _tc_layer1(x, W0, b0, Wc1, dis_b):
    def kfn(x_ref, w0_ref, b0_ref, wc1_ref, dis_ref, o_ref):
        i = pl.program_id(0)
        h0 = jnp.dot(x_ref[...], w0_ref[...], preferred_element_type=jnp.float32)
        h0 = jnp.maximum(h0 + b0_ref[...], 0.0)
        u = jnp.dot(h0, wc1_ref[...], preferred_element_type=jnp.float32)
        row = lax.broadcasted_iota(jnp.int32, (BM, D), 0) + i * BM
        o_ref[...] = jnp.where(row < N, dis_ref[...] * u, 0.0)

    return pl.pallas_call(
        kfn,
        grid=(GRID,),
        in_specs=[
            pl.BlockSpec((BM, D), lambda i: (i, 0)),
            pl.BlockSpec((D, D), lambda i: (0, 0)),
            pl.BlockSpec((1, D), lambda i: (0, 0)),
            pl.BlockSpec((D, D), lambda i: (0, 0)),
            pl.BlockSpec((BM, D), lambda i: (i, 0)),
        ],
        out_specs=pl.BlockSpec((BM, D), lambda i: (i, 0)),
        out_shape=jax.ShapeDtypeStruct((NR, D), jnp.float32),
    )(x, W0, b0.reshape(1, D), Wc1, dis_b)


def _tc_layer2(aggp, h1p, dis_b, bc1, Wc2):
    def kfn(p0_ref, p1_ref, h1_ref, dis_ref, b1_ref, wc2_ref, o_ref):
        i = pl.program_id(0)
        t = dis_ref[...] * (p0_ref[...] + p1_ref[...] + h1_ref[...]) + b1_ref[...]
        out1 = jnp.maximum(t, 0.0)
        u = jnp.dot(out1, wc2_ref[...], preferred_element_type=jnp.float32)
        row = lax.broadcasted_iota(jnp.int32, (BM, D), 0) + i * BM
        o_ref[...] = jnp.where(row < N, dis_ref[...] * u, 0.0)

    return pl.pallas_call(
        kfn,
        grid=(GRID,),
        in_specs=[
            pl.BlockSpec((BM, D), lambda i: (i, 0)),
            pl.BlockSpec((BM, D), lambda i: (i + GRID, 0)),
            pl.BlockSpec((BM, D), lambda i: (i, 0)),
            pl.BlockSpec((BM, D), lambda i: (i, 0)),
            pl.BlockSpec((1, D), lambda i: (0, 0)),
            pl.BlockSpec((D, D), lambda i: (0, 0)),
        ],
        out_specs=pl.BlockSpec((BM, D), lambda i: (i, 0)),
        out_shape=jax.ShapeDtypeStruct((NR, D), jnp.float32),
    )(aggp, aggp, h1p, dis_b, bc1.reshape(1, D), Wc2)


def _tc_out(aggp, h2p, dis_b, bc2):
    def kfn(q0_ref, q1_ref, h2_ref, dis_ref, b2_ref, o_ref):
        t = dis_ref[...] * (q0_ref[...] + q1_ref[...] + h2_ref[...]) + b2_ref[...]
        o_ref[...] = jnp.maximum(t, 0.0)

    return pl.pallas_call(
        kfn,
        grid=(GRID,),
        in_specs=[
            pl.BlockSpec((BM, D), lambda i: (i, 0)),
            pl.BlockSpec((BM, D), lambda i: (i + GRID, 0)),
            pl.BlockSpec((BM, D), lambda i: (i, 0)),
            pl.BlockSpec((BM, D), lambda i: (i, 0)),
            pl.BlockSpec((1, D), lambda i: (0, 0)),
        ],
        out_specs=pl.BlockSpec((BM, D), lambda i: (i, 0)),
        out_shape=jax.ShapeDtypeStruct((N, D), jnp.float32),
    )(aggp, aggp, h2p, dis_b, bc2.reshape(1, D))


def kernel(input, edge_index, W0, b0, Wc1, bc1, Wc2, bc2):
    x = input
    e = edge_index.shape[1]
    ei = edge_index.astype(jnp.int32)
    epw = -(-e // (NW * WIN)) * WIN
    npad = NW * epw - e
    nwin = epw // WIN
    pad_idx = N + (jnp.arange(npad, dtype=jnp.int32) % (NR - N))
    src_pad = jnp.concatenate([ei[0], pad_idx])
    dst_pad = jnp.concatenate([ei[1], pad_idx])

    degp = _sc_deg(dst_pad, epw, nwin)
    dis_b = _tc_dis(degp)
    h1p = _tc_layer1(x, W0, b0, Wc1, dis_b)
    agg1 = _sc_scatter(h1p, src_pad, dst_pad, epw, nwin)
    h2p = _tc_layer2(agg1, h1p, dis_b, bc1, Wc2)
    agg2 = _sc_scatter(h2p, src_pad, dst_pad, epw, nwin)
    return _tc_out(agg2, h2p, dis_b, bc2)


# speedup vs baseline: 19.0812x; 1.2682x over previous
import functools

import jax
import jax.numpy as jnp
from jax import lax
from jax.experimental import pallas as pl
from jax.experimental.pallas import tpu as pltpu
from jax.experimental.pallas import tpu_sc as plsc

N = 10000
D = 128
NR = 10240
NC = 2
NS = 16
NW = NC * NS
WIN = 128
RPT = NR // NS
BM = 1024
GRID = NR // BM


def _fill_2d(ref, rows, value):
    v = jnp.full((16,), value, jnp.float32)
    cols = ref.shape[1]

    def body(r, _):
        for j in range(cols // 16):
            ref[r, pl.ds(j * 16, 16)] = v
        return 0

    lax.fori_loop(0, rows, body, 0)


def _sc_deg(dst_pad, epw, nwin):
    @functools.partial(
        pl.kernel,
        mesh=plsc.VectorSubcoreMesh(core_axis_name="c", subcore_axis_name="s"),
        out_type=jax.ShapeDtypeStruct((NC * NR, D), jnp.float32),
        scratch_types=[
            pltpu.VMEM((WIN,), jnp.int32),
            pltpu.VMEM((WIN,), jnp.int32),
            pltpu.VMEM((WIN, D), jnp.float32),
            pltpu.VMEM_SHARED((NR, D), jnp.float32),
            pltpu.SemaphoreType.DMA,
            pltpu.SemaphoreType.DMA,
        ],
    )
    def k(dst_hbm, out_hbm, didx0, didx1, ones_v, acc, isem0, isem1):
        c = lax.axis_index("c")
        s = lax.axis_index("s")
        wid = c * NS + s
        _fill_2d(ones_v, WIN, 0.0)
        for j in range(RPT // WIN):
            pltpu.sync_copy(ones_v, acc.at[pl.ds(s * RPT + j * WIN, WIN)])
        _fill_2d(ones_v, WIN, 1.0)
        plsc.subcore_barrier()
        ebase = wid * epw
        last = nwin - 1

        def win(w, _):
            pltpu.sync_copy(dst_hbm.at[pl.ds(ebase + w * WIN, WIN)], didx0)
            pltpu.sync_copy(ones_v, acc.at[didx0], add=True)
            return 0

        lax.fori_loop(0, nwin, win, 0)
        plsc.subcore_barrier()
        pltpu.sync_copy(acc.at[pl.ds(s * RPT, RPT)],
                        out_hbm.at[pl.ds(c * NR + s * RPT, RPT)])

    return k(dst_pad)


def _sc_scatter(h_pad, src_pad, dst_pad, epw, nwin):
    @functools.partial(
        pl.kernel,
        mesh=plsc.VectorSubcoreMesh(core_axis_name="c", subcore_axis_name="s"),
        out_type=jax.ShapeDtypeStruct((NC * NR, D), jnp.float32),
        scratch_types=[
            pltpu.VMEM((WIN,), jnp.int32),
            pltpu.VMEM((WIN,), jnp.int32),
            pltpu.VMEM((WIN,), jnp.int32),
            pltpu.VMEM((WIN,), jnp.int32),
            pltpu.VMEM((WIN,), jnp.int32),
            pltpu.VMEM((WIN,), jnp.int32),
            pltpu.VMEM((WIN, D), jnp.float32),
            pltpu.VMEM_SHARED((NR, D), jnp.float32),
            pltpu.SemaphoreType.DMA,
            pltpu.SemaphoreType.DMA,
            pltpu.SemaphoreType.DMA,
            pltpu.SemaphoreType.DMA,
        ],
    )
    def k(h_hbm, src_hbm, dst_hbm, out_hbm, sidx0, didx0, sidx1, didx1,
          sidx2, didx2, rows, acc, gsem, isem0, isem1, isem2):
        c = lax.axis_index("c")
        s = lax.axis_index("s")
        wid = c * NS + s
        _fill_2d(rows, WIN, 0.0)
        for j in range(RPT // WIN):
            pltpu.sync_copy(rows, acc.at[pl.ds(s * RPT + j * WIN, WIN)])
        plsc.subcore_barrier()

        last = nwin - 1
        ebase = wid * epw

        def idx_start(w, sb, db, sem):
            off = ebase + jnp.minimum(w, last) * WIN
            pltpu.async_copy(src_hbm.at[pl.ds(off, WIN)], sb, sem)
            pltpu.async_copy(dst_hbm.at[pl.ds(off, WIN)], db, sem)

        def idx_wait(sb, db, sem):
            pltpu.make_async_copy(src_hbm.at[pl.ds(0, WIN)], sb, sem).wait()
            pltpu.make_async_copy(dst_hbm.at[pl.ds(0, WIN)], db, sem).wait()

        idx_start(0, sidx0, didx0, isem0)
        idx_wait(sidx0, didx0, isem0)
        idx_start(1, sidx1, didx1, isem1)
        idx_wait(sidx1, didx1, isem1)

        def one(w, sa, da, sema, sb, db, semb):
            pltpu.async_copy(h_hbm.at[sa], rows, gsem).wait()
            idx_start(w + 2, sb, db, semb)
            pltpu.sync_copy(rows, acc.at[da], add=True)
            idx_wait(sb, db, semb)

        def win(i, _):
            w = i * 3
            one(w, sidx0, didx0, isem0, sidx2, didx2, isem2)
            one(w + 1, sidx1, didx1, isem1, sidx0, didx0, isem0)
            one(w + 2, sidx2, didx2, isem2, sidx1, didx1, isem1)
            return 0

        nfull = nwin // 3
        lax.fori_loop(0, nfull, win, 0)
        ring = [(sidx0, didx0, isem0), (sidx1, didx1, isem1),
                (sidx2, didx2, isem2)]
        for t in range(nwin - nfull * 3):
            sa, da, _ = ring[t % 3]
            pltpu.async_copy(h_hbm.at[sa], rows, gsem).wait()
            pltpu.sync_copy(rows, acc.at[da], add=True)
        plsc.subcore_barrier()
        pltpu.sync_copy(acc.at[pl.ds(s * RPT, RPT)],
                        out_hbm.at[pl.ds(c * NR + s * RPT, RPT)])

    return k(h_pad, src_pad, dst_pad)


def _tc_dis(deg_parts):
    def kfn(d0_ref, d1_ref, oref):
        deg = d0_ref[:, 0:1] + d1_ref[:, 0:1] + 1.0
        oref[...] = jnp.broadcast_to(lax.rsqrt(deg), (BM, D))

    return pl.pallas_call(
        kfn,
        grid=(GRID,),
        in_specs=[
            pl.BlockSpec((BM, D), lambda i: (i, 0)),
            pl.BlockSpec((BM, D), lambda i: (i + GRID, 0)),
        ],
        out_specs=pl.BlockSpec((BM, D), lambda i: (i, 0)),
        out_shape=jax.ShapeDtypeStruct((NR, D), jnp.float32),
    )(deg_parts, deg_parts)


def _tc_layer1(x, W0, b0, Wc1, dis_b):
    def kfn(x_ref, w0_ref, b0_ref, wc1_ref, dis_ref, o_ref):
        i = pl.program_id(0)
        h0 = jnp.dot(x_ref[...], w0_ref[...], preferred_element_type=jnp.float32)
        h0 = jnp.maximum(h0 + b0_ref[...], 0.0)
        u = jnp.dot(h0, wc1_ref[...], preferred_element_type=jnp.float32)
        row = lax.broadcasted_iota(jnp.int32, (BM, D), 0) + i * BM
        o_ref[...] = jnp.where(row < N, dis_ref[...] * u, 0.0)

    return pl.pallas_call(
        kfn,
        grid=(GRID,),
        in_specs=[
            pl.BlockSpec((BM, D), lambda i: (i, 0)),
            pl.BlockSpec((D, D), lambda i: (0, 0)),
            pl.BlockSpec((1, D), lambda i: (0, 0)),
            pl.BlockSpec((D, D), lambda i: (0, 0)),
            pl.BlockSpec((BM, D), lambda i: (i, 0)),
        ],
        out_specs=pl.BlockSpec((BM, D), lambda i: (i, 0)),
        out_shape=jax.ShapeDtypeStruct((NR, D), jnp.float32),
    )(x, W0, b0.reshape(1, D), Wc1, dis_b)


def _tc_layer2(aggp, h1p, dis_b, bc1, Wc2):
    def kfn(p0_ref, p1_ref, h1_ref, dis_ref, b1_ref, wc2_ref, o_ref):
        i = pl.program_id(0)
        t = dis_ref[...] * (p0_ref[...] + p1_ref[...] + h1_ref[...]) + b1_ref[...]
        out1 = jnp.maximum(t, 0.0)
        u = jnp.dot(out1, wc2_ref[...], preferred_element_type=jnp.float32)
        row = lax.broadcasted_iota(jnp.int32, (BM, D), 0) + i * BM
        o_ref[...] = jnp.where(row < N, dis_ref[...] * u, 0.0)

    return pl.pallas_call(
        kfn,
        grid=(GRID,),
        in_specs=[
            pl.BlockSpec((BM, D), lambda i: (i, 0)),
            pl.BlockSpec((BM, D), lambda i: (i + GRID, 0)),
            pl.BlockSpec((BM, D), lambda i: (i, 0)),
            pl.BlockSpec((BM, D), lambda i: (i, 0)),
            pl.BlockSpec((1, D), lambda i: (0, 0)),
            pl.BlockSpec((D, D), lambda i: (0, 0)),
        ],
        out_specs=pl.BlockSpec((BM, D), lambda i: (i, 0)),
        out_shape=jax.ShapeDtypeStruct((NR, D), jnp.float32),
    )(aggp, aggp, h1p, dis_b, bc1.reshape(1, D), Wc2)


def _tc_out(aggp, h2p, dis_b, bc2):
    def kfn(q0_ref, q1_ref, h2_ref, dis_ref, b2_ref, o_ref):
        t = dis_ref[...] * (q0_ref[...] + q1_ref[...] + h2_ref[...]) + b2_ref[...]
        o_ref[...] = jnp.maximum(t, 0.0)

    return pl.pallas_call(
        kfn,
        grid=(GRID,),
        in_specs=[
            pl.BlockSpec((BM, D), lambda i: (i, 0)),
            pl.BlockSpec((BM, D), lambda i: (i + GRID, 0)),
            pl.BlockSpec((BM, D), lambda i: (i, 0)),
            pl.BlockSpec((BM, D), lambda i: (i, 0)),
            pl.BlockSpec((1, D), lambda i: (0, 0)),
        ],
        out_specs=pl.BlockSpec((BM, D), lambda i: (i, 0)),
        out_shape=jax.ShapeDtypeStruct((N, D), jnp.float32),
    )(aggp, aggp, h2p, dis_b, bc2.reshape(1, D))


def kernel(input, edge_index, W0, b0, Wc1, bc1, Wc2, bc2):
    x = input
    e = edge_index.shape[1]
    ei = edge_index.astype(jnp.int32)
    epw = -(-e // (NW * WIN)) * WIN
    npad = NW * epw - e
    nwin = epw // WIN
    pad_idx = N + (jnp.arange(npad, dtype=jnp.int32) % (NR - N))
    src_pad = jnp.concatenate([ei[0], pad_idx])
    dst_pad = jnp.concatenate([ei[1], pad_idx])
    src2d = src_pad.reshape(NW, nwin, WIN)
    dst2d = dst_pad.reshape(NW, nwin, WIN)

    degp = _sc_deg(dst_pad, epw, nwin)
    dis_b = _tc_dis(degp)
    h1p = _tc_layer1(x, W0, b0, Wc1, dis_b)
    agg1 = _sc_scatter(h1p, src_pad, dst_pad, epw, nwin)
    h2p = _tc_layer2(agg1, h1p, dis_b, bc1, Wc2)
    agg2 = _sc_scatter(h2p, src_pad, dst_pad, epw, nwin)
    return _tc_out(agg2, h2p, dis_b, bc2)


# speedup vs baseline: 22.5997x; 1.1844x over previous
import functools

import jax
import jax.numpy as jnp
from jax import lax
from jax.experimental import pallas as pl
from jax.experimental.pallas import tpu as pltpu
from jax.experimental.pallas import tpu_sc as plsc

N = 10000
D = 128
NR = 10240
NC = 2
NS = 16
NW = NC * NS
WIN = 128
RPT = NR // NS
BM = 1024
GRID = NR // BM


def _fill_2d(ref, rows, value):
    v = jnp.full((16,), value, jnp.float32)
    cols = ref.shape[1]

    def body(r, _):
        for j in range(cols // 16):
            ref[r, pl.ds(j * 16, 16)] = v
        return 0

    lax.fori_loop(0, rows, body, 0)


def _sc_deg(dst_pad, epw, nwin):
    @functools.partial(
        pl.kernel,
        mesh=plsc.VectorSubcoreMesh(core_axis_name="c", subcore_axis_name="s"),
        out_type=jax.ShapeDtypeStruct((NC * NR,), jnp.float32),
        compiler_params=pltpu.CompilerParams(needs_layout_passes=False),
        scratch_types=[
            pltpu.VMEM((epw,), jnp.int32),
            pltpu.VMEM((NR,), jnp.float32),
            pltpu.VMEM((NS, RPT), jnp.float32),
            pltpu.VMEM_SHARED((NS, NR), jnp.float32),
        ],
    )
    def k(dst_hbm, out_hbm, didx, hist, buf, shared):
        c = lax.axis_index("c")
        s = lax.axis_index("s")
        wid = c * NS + s
        zero = jnp.zeros((16,), jnp.float32)

        def z(i, _):
            hist[pl.ds(i * 16, 16)] = zero
            return 0

        lax.fori_loop(0, NR // 16, z, 0)
        pltpu.sync_copy(dst_hbm.at[pl.ds(wid * epw, epw)], didx)
        ones = jnp.full((16,), 1.0, jnp.float32)

        def grp(g, _):
            kk = didx[pl.ds(g * 16, 16)]
            plsc.addupdate_scatter(hist, [kk], ones)
            return 0

        lax.fori_loop(0, epw // 16, grp, 0)
        pltpu.sync_copy(hist, shared.at[s])
        plsc.subcore_barrier()
        pltpu.sync_copy(shared.at[:, pl.ds(s * RPT, RPT)], buf)

        def colsum(j, _):
            a = buf[0, pl.ds(j * 16, 16)]
            for r in range(1, NS):
                a = a + buf[r, pl.ds(j * 16, 16)]
            hist[pl.ds(j * 16, 16)] = a
            return 0

        lax.fori_loop(0, RPT // 16, colsum, 0)
        pltpu.sync_copy(hist.at[pl.ds(0, RPT)],
                        out_hbm.at[pl.ds(c * NR + s * RPT, RPT)])

    return k(dst_pad)


def _sc_scatter(h_pad, src_pad, dst_pad, epw, nwin):
    @functools.partial(
        pl.kernel,
        mesh=plsc.VectorSubcoreMesh(core_axis_name="c", subcore_axis_name="s"),
        out_type=jax.ShapeDtypeStruct((NC * NR, D), jnp.float32),
        scratch_types=[
            pltpu.VMEM((WIN,), jnp.int32),
            pltpu.VMEM((WIN,), jnp.int32),
            pltpu.VMEM((WIN,), jnp.int32),
            pltpu.VMEM((WIN,), jnp.int32),
            pltpu.VMEM((WIN,), jnp.int32),
            pltpu.VMEM((WIN,), jnp.int32),
            pltpu.VMEM((WIN, D), jnp.float32),
            pltpu.VMEM_SHARED((NR, D), jnp.float32),
            pltpu.SemaphoreType.DMA,
            pltpu.SemaphoreType.DMA,
            pltpu.SemaphoreType.DMA,
            pltpu.SemaphoreType.DMA,
        ],
    )
    def k(h_hbm, src_hbm, dst_hbm, out_hbm, sidx0, didx0, sidx1, didx1,
          sidx2, didx2, rows, acc, gsem, isem0, isem1, isem2):
        c = lax.axis_index("c")
        s = lax.axis_index("s")
        wid = c * NS + s
        _fill_2d(rows, WIN, 0.0)
        for j in range(RPT // WIN):
            pltpu.sync_copy(rows, acc.at[pl.ds(s * RPT + j * WIN, WIN)])
        plsc.subcore_barrier()

        last = nwin - 1
        ebase = wid * epw

        def idx_start(w, sb, db, sem):
            off = ebase + jnp.minimum(w, last) * WIN
            pltpu.async_copy(src_hbm.at[pl.ds(off, WIN)], sb, sem)
            pltpu.async_copy(dst_hbm.at[pl.ds(off, WIN)], db, sem)

        def idx_wait(sb, db, sem):
            pltpu.make_async_copy(src_hbm.at[pl.ds(0, WIN)], sb, sem).wait()
            pltpu.make_async_copy(dst_hbm.at[pl.ds(0, WIN)], db, sem).wait()

        idx_start(0, sidx0, didx0, isem0)
        idx_wait(sidx0, didx0, isem0)
        idx_start(1, sidx1, didx1, isem1)
        idx_wait(sidx1, didx1, isem1)

        def one(w, sa, da, sema, sb, db, semb):
            pltpu.async_copy(h_hbm.at[sa], rows, gsem).wait()
            idx_start(w + 2, sb, db, semb)
            pltpu.sync_copy(rows, acc.at[da], add=True)
            idx_wait(sb, db, semb)

        def win(i, _):
            w = i * 3
            one(w, sidx0, didx0, isem0, sidx2, didx2, isem2)
            one(w + 1, sidx1, didx1, isem1, sidx0, didx0, isem0)
            one(w + 2, sidx2, didx2, isem2, sidx1, didx1, isem1)
            return 0

        nfull = nwin // 3
        lax.fori_loop(0, nfull, win, 0)
        ring = [(sidx0, didx0, isem0), (sidx1, didx1, isem1),
                (sidx2, didx2, isem2)]
        for t in range(nwin - nfull * 3):
            sa, da, _ = ring[t % 3]
            pltpu.async_copy(h_hbm.at[sa], rows, gsem).wait()
            pltpu.sync_copy(rows, acc.at[da], add=True)
        plsc.subcore_barrier()
        pltpu.sync_copy(acc.at[pl.ds(s * RPT, RPT)],
                        out_hbm.at[pl.ds(c * NR + s * RPT, RPT)])

    return k(h_pad, src_pad, dst_pad)


def _tc_dis(d0, d1):
    def kfn(d0_ref, d1_ref, oref):
        deg = d0_ref[...] + d1_ref[...] + 1.0
        oref[...] = jnp.broadcast_to(lax.rsqrt(deg), (BM, D))

    return pl.pallas_call(
        kfn,
        grid=(GRID,),
        in_specs=[
            pl.BlockSpec((BM, 1), lambda i: (i, 0)),
            pl.BlockSpec((BM, 1), lambda i: (i, 0)),
        ],
        out_specs=pl.BlockSpec((BM, D), lambda i: (i, 0)),
        out_shape=jax.ShapeDtypeStruct((NR, D), jnp.float32),
    )(d0, d1)


def _tc_layer1(x, W0, b0, Wc1, dis_b):
    def kfn(x_ref, w0_ref, b0_ref, wc1_ref, dis_ref, o_ref):
        i = pl.program_id(0)
        h0 = jnp.dot(x_ref[...], w0_ref[...], preferred_element_type=jnp.float32)
        h0 = jnp.maximum(h0 + b0_ref[...], 0.0)
        u = jnp.dot(h0, wc1_ref[...], preferred_element_type=jnp.float32)
        row = lax.broadcasted_iota(jnp.int32, (BM, D), 0) + i * BM
        o_ref[...] = jnp.where(row < N, dis_ref[...] * u, 0.0)

    return pl.pallas_call(
        kfn,
        grid=(GRID,),
        in_specs=[
            pl.BlockSpec((BM, D), lambda i: (i, 0)),
            pl.BlockSpec((D, D), lambda i: (0, 0)),
            pl.BlockSpec((1, D), lambda i: (0, 0)),
            pl.BlockSpec((D, D), lambda i: (0, 0)),
            pl.BlockSpec((BM, D), lambda i: (i, 0)),
        ],
        out_specs=pl.BlockSpec((BM, D), lambda i: (i, 0)),
        out_shape=jax.ShapeDtypeStruct((NR, D), jnp.float32),
    )(x, W0, b0.reshape(1, D), Wc1, dis_b)


def _tc_layer2(aggp, h1p, dis_b, bc1, Wc2):
    def kfn(p0_ref, p1_ref, h1_ref, dis_ref, b1_ref, wc2_ref, o_ref):
        i = pl.program_id(0)
        t = dis_ref[...] * (p0_ref[...] + p1_ref[...] + h1_ref[...]) + b1_ref[...]
        out1 = jnp.maximum(t, 0.0)
        u = jnp.dot(out1, wc2_ref[...], preferred_element_type=jnp.float32)
        row = lax.broadcasted_iota(jnp.int32, (BM, D), 0) + i * BM
        o_ref[...] = jnp.where(row < N, dis_ref[...] * u, 0.0)

    return pl.pallas_call(
        kfn,
        grid=(GRID,),
        in_specs=[
            pl.BlockSpec((BM, D), lambda i: (i, 0)),
            pl.BlockSpec((BM, D), lambda i: (i + GRID, 0)),
            pl.BlockSpec((BM, D), lambda i: (i, 0)),
            pl.BlockSpec((BM, D), lambda i: (i, 0)),
            pl.BlockSpec((1, D), lambda i: (0, 0)),
            pl.BlockSpec((D, D), lambda i: (0, 0)),
        ],
        out_specs=pl.BlockSpec((BM, D), lambda i: (i, 0)),
        out_shape=jax.ShapeDtypeStruct((NR, D), jnp.float32),
    )(aggp, aggp, h1p, dis_b, bc1.reshape(1, D), Wc2)


def _tc_out(aggp, h2p, dis_b, bc2):
    def kfn(q0_ref, q1_ref, h2_ref, dis_ref, b2_ref, o_ref):
        t = dis_ref[...] * (q0_ref[...] + q1_ref[...] + h2_ref[...]) + b2_ref[...]
        o_ref[...] = jnp.maximum(t, 0.0)

    return pl.pallas_call(
        kfn,
        grid=(GRID,),
        in_specs=[
            pl.BlockSpec((BM, D), lambda i: (i, 0)),
            pl.BlockSpec((BM, D), lambda i: (i + GRID, 0)),
            pl.BlockSpec((BM, D), lambda i: (i, 0)),
            pl.BlockSpec((BM, D), lambda i: (i, 0)),
            pl.BlockSpec((1, D), lambda i: (0, 0)),
        ],
        out_specs=pl.BlockSpec((BM, D), lambda i: (i, 0)),
        out_shape=jax.ShapeDtypeStruct((N, D), jnp.float32),
    )(aggp, aggp, h2p, dis_b, bc2.reshape(1, D))


def kernel(input, edge_index, W0, b0, Wc1, bc1, Wc2, bc2):
    x = input
    e = edge_index.shape[1]
    ei = edge_index.astype(jnp.int32)
    epw = -(-e // (NW * WIN)) * WIN
    npad = NW * epw - e
    nwin = epw // WIN
    pad_idx = N + (jnp.arange(npad, dtype=jnp.int32) % (NR - N))
    src_pad = jnp.concatenate([ei[0], pad_idx])
    dst_pad = jnp.concatenate([ei[1], pad_idx])
    src2d = src_pad.reshape(NW, nwin, WIN)
    dst2d = dst_pad.reshape(NW, nwin, WIN)

    degp = _sc_deg(dst_pad, epw, nwin)
    dis_b = _tc_dis(degp[:NR].reshape(NR, 1), degp[NR:].reshape(NR, 1))
    h1p = _tc_layer1(x, W0, b0, Wc1, dis_b)
    agg1 = _sc_scatter(h1p, src_pad, dst_pad, epw, nwin)
    h2p = _tc_layer2(agg1, h1p, dis_b, bc1, Wc2)
    agg2 = _sc_scatter(h2p, src_pad, dst_pad, epw, nwin)
    return _tc_out(agg2, h2p, dis_b, bc2)
